# Initial kernel scaffold; baseline (speedup 1.0000x reference)
import jax
import jax.numpy as jnp
from jax.experimental import pallas as pl


def kernel(pred_pointclouds, gt_pointclouds):
    raise NotImplementedError("write your pallas kernel here")


# speedup vs baseline: 1.9135x; 1.9135x over previous
import functools

import jax
import jax.numpy as jnp
from jax.experimental import pallas as pl


_N = 4096
_TILE = 512
_K = _N // 2
_WEIGHT = 3.0


def _top_half_mean(x, xb, k):
    def body(i, m):
        cand = m | jnp.left_shift(jnp.int32(1), jnp.int32(30) - i)
        cnt = jnp.sum(jnp.where(xb >= cand, jnp.int32(1), jnp.int32(0)))
        return jax.lax.select(cnt >= k, cand, m)

    m = jax.lax.fori_loop(0, 31, body, jnp.int32(0))
    t = jax.lax.bitcast_convert_type(m, jnp.float32)
    gt_mask = xb > m
    cnt_gt = jnp.sum(jnp.where(gt_mask, jnp.int32(1), jnp.int32(0)))
    sum_gt = jnp.sum(jnp.where(gt_mask, x, jnp.float32(0.0)))
    total = sum_gt + (jnp.int32(k) - cnt_gt).astype(jnp.float32) * t
    return total / jnp.float32(k)


def _chamfer_kernel(pred_ref, gtt_ref, out_ref):
    b = pl.program_id(0)

    gT = gtt_ref[0]
    ng2 = jnp.sum(gT * gT, axis=0, keepdims=True)

    colmin = jnp.full((1, _N), jnp.inf, dtype=jnp.float32)
    rowmins = []
    for i in range(_N // _TILE):
        p_tile = pred_ref[0, i * _TILE:(i + 1) * _TILE, :]
        cross = jnp.dot(p_tile, gT, preferred_element_type=jnp.float32)
        np2 = jnp.sum(p_tile * p_tile, axis=1, keepdims=True)
        d = np2 + ng2 - 2.0 * cross
        d = jnp.maximum(d, 1e-12)
        rowmins.append(jnp.min(d, axis=1, keepdims=True))
        colmin = jnp.minimum(colmin, jnp.min(d, axis=0, keepdims=True))

    rm = jnp.concatenate(rowmins, axis=1)
    d1 = jnp.sqrt(rm)
    d2 = jnp.sqrt(colmin)

    mean1 = jnp.sum(d1) / jnp.float32(_N)
    mean2 = jnp.sum(d2) / jnp.float32(_N)

    d1b = jax.lax.bitcast_convert_type(d1, jnp.int32)
    d2b = jax.lax.bitcast_convert_type(d2, jnp.int32)
    w1 = _top_half_mean(d1, d1b, _K)
    w2 = _top_half_mean(d2, d2b, _K)

    loss = mean1 + mean2 + _WEIGHT * (w1 + w2)

    @pl.when(b == 0)
    def _():
        out_ref[:, :] = jnp.zeros((1, 1), dtype=jnp.float32)

    out_ref[:, :] = out_ref[:, :] + jnp.full((1, 1), 0.25, jnp.float32) * loss


def kernel(pred_pointclouds, gt_pointclouds):
    B, N, _ = pred_pointclouds.shape
    gtT = jnp.transpose(gt_pointclouds, (0, 2, 1))

    out = pl.pallas_call(
        _chamfer_kernel,
        grid=(B,),
        in_specs=[
            pl.BlockSpec((1, N, 3), lambda b: (b, 0, 0)),
            pl.BlockSpec((1, 3, N), lambda b: (b, 0, 0)),
        ],
        out_specs=pl.BlockSpec((1, 1), lambda b: (0, 0)),
        out_shape=jax.ShapeDtypeStruct((1, 1), jnp.float32),
    )(pred_pointclouds, gtT)
    return out[0, 0]


# speedup vs baseline: 2.1658x; 1.1319x over previous
import jax
import jax.numpy as jnp
from jax.experimental import pallas as pl


_N = 4096
_TILE = 512
_K = _N // 2
_WEIGHT = 3.0
_B = 4


def _nn_row(X, n2col, YTm2, n2row):
    colmin = jnp.full((1, _N), jnp.inf, dtype=jnp.float32)
    for i in range(_N // _TILE):
        sl = slice(i * _TILE, (i + 1) * _TILE)
        cross = jnp.dot(X[sl, :], YTm2,
                        preferred_element_type=jnp.float32)
        e = cross + n2col[sl, :]
        colmin = jnp.minimum(colmin, jnp.min(e, axis=0, keepdims=True))
    return jnp.maximum(colmin + n2row, 1e-12)


def _chamfer_kernel(pred_ref, predt_ref, gt_ref, gtt_ref, out_ref):
    rows = []
    for b in range(_B):
        p = pred_ref[b]
        g = gt_ref[b]
        pT = predt_ref[b]
        gT = gtt_ref[b]
        np2c = jnp.sum(p * p, axis=1, keepdims=True)
        ng2c = jnp.sum(g * g, axis=1, keepdims=True)
        np2r = jnp.sum(pT * pT, axis=0, keepdims=True)
        ng2r = jnp.sum(gT * gT, axis=0, keepdims=True)
        d1sq = _nn_row(g, ng2c, -2.0 * pT, np2r)
        d2sq = _nn_row(p, np2c, -2.0 * gT, ng2r)
        rows.append(jnp.sqrt(d1sq))
        rows.append(jnp.sqrt(d2sq))

    D = jnp.concatenate(rows, axis=0)
    Db = jax.lax.bitcast_convert_type(D, jnp.int32)

    def body(i, m):
        cand = m | jnp.left_shift(jnp.int32(1), jnp.int32(30) - i)
        cnt = jnp.sum(jnp.where(Db >= cand, jnp.int32(1), jnp.int32(0)),
                      axis=1, keepdims=True)
        return jnp.where(cnt >= _K, cand, m)

    m = jax.lax.fori_loop(0, 31, body, jnp.zeros((2 * _B, 1), jnp.int32))
    t = jax.lax.bitcast_convert_type(m, jnp.float32)
    gt_mask = Db > m
    cnt_gt = jnp.sum(jnp.where(gt_mask, jnp.int32(1), jnp.int32(0)),
                     axis=1, keepdims=True)
    sum_gt = jnp.sum(jnp.where(gt_mask, D, jnp.float32(0.0)),
                     axis=1, keepdims=True)
    w = (sum_gt + (jnp.int32(_K) - cnt_gt).astype(jnp.float32) * t) / _K

    s_means = jnp.sum(D, axis=1, keepdims=True) / jnp.float32(_N)
    total = jnp.sum(s_means + _WEIGHT * w)
    out_ref[:, :] = jnp.full((1, 1), 1.0 / _B, jnp.float32) * total


def kernel(pred_pointclouds, gt_pointclouds):
    predT = jnp.transpose(pred_pointclouds, (0, 2, 1))
    gtT = jnp.transpose(gt_pointclouds, (0, 2, 1))

    out = pl.pallas_call(
        _chamfer_kernel,
        out_shape=jax.ShapeDtypeStruct((1, 1), jnp.float32),
    )(pred_pointclouds, predT, gt_pointclouds, gtT)
    return out[0, 0]


# speedup vs baseline: 3.5640x; 1.6456x over previous
import jax
import jax.numpy as jnp
from jax.experimental import pallas as pl


_N = 4096
_TILE = 512
_K = _N // 2
_WEIGHT = 3.0
_B = 4


def _chamfer_kernel(pred_ref, gtt_ref, out_ref):
    rows = []
    for b in range(_B):
        p = pred_ref[b]
        gT = gtt_ref[b]
        np2c = jnp.sum(p * p, axis=1, keepdims=True)
        ng2r = jnp.sum(gT * gT, axis=0, keepdims=True)
        gTm2 = -2.0 * gT

        colmin = jnp.full((1, _N), jnp.inf, dtype=jnp.float32)
        d1_pieces = []
        for i in range(_N // _TILE):
            sl = slice(i * _TILE, (i + 1) * _TILE)
            cross = jnp.dot(p[sl, :], gTm2,
                            preferred_element_type=jnp.float32)
            e = cross + np2c[sl, :]
            colmin = jnp.minimum(colmin, jnp.min(e, axis=0, keepdims=True))
            e4 = e + ng2r
            rm = jnp.min(e4, axis=1, keepdims=True)
            d1_pieces.append(jax.lax.transpose(rm, (1, 0)))

        d1sq = jnp.maximum(jnp.concatenate(d1_pieces, axis=1), 1e-12)
        d2sq = jnp.maximum(colmin + ng2r, 1e-12)
        rows.append(jnp.sqrt(d1sq))
        rows.append(jnp.sqrt(d2sq))

    D = jnp.concatenate(rows, axis=0)
    Db = jax.lax.bitcast_convert_type(D, jnp.int32)

    def body(i, m):
        cand = m | jnp.left_shift(jnp.int32(1), jnp.int32(30) - i)
        cnt = jnp.sum(jnp.where(Db >= cand, jnp.int32(1), jnp.int32(0)),
                      axis=1, keepdims=True)
        return jnp.where(cnt >= _K, cand, m)

    m = jax.lax.fori_loop(0, 31, body, jnp.zeros((2 * _B, 1), jnp.int32))
    t = jax.lax.bitcast_convert_type(m, jnp.float32)
    gt_mask = Db > m
    cnt_gt = jnp.sum(jnp.where(gt_mask, jnp.int32(1), jnp.int32(0)),
                     axis=1, keepdims=True)
    sum_gt = jnp.sum(jnp.where(gt_mask, D, jnp.float32(0.0)),
                     axis=1, keepdims=True)
    w = (sum_gt + (jnp.int32(_K) - cnt_gt).astype(jnp.float32) * t) / _K

    s_means = jnp.sum(D, axis=1, keepdims=True) / jnp.float32(_N)
    total = jnp.sum(s_means + _WEIGHT * w)
    out_ref[:, :] = jnp.full((1, 1), 1.0 / _B, jnp.float32) * total


def kernel(pred_pointclouds, gt_pointclouds):
    gtT = jnp.transpose(gt_pointclouds, (0, 2, 1))

    out = pl.pallas_call(
        _chamfer_kernel,
        out_shape=jax.ShapeDtypeStruct((1, 1), jnp.float32),
    )(pred_pointclouds, gtT)
    return out[0, 0]


# speedup vs baseline: 3.5835x; 1.0055x over previous
import jax
import jax.numpy as jnp
from jax.experimental import pallas as pl


_N = 4096
_TILE = 1024
_K = _N // 2
_WEIGHT = 3.0
_B = 4


def _chamfer_kernel(pred_ref, gtt_ref, out_ref):
    rows = []
    for b in range(_B):
        p = pred_ref[b]
        gT = gtt_ref[b]
        np2c = jnp.sum(p * p, axis=1, keepdims=True)
        ng2r = jnp.sum(gT * gT, axis=0, keepdims=True)
        gTm2 = -2.0 * gT

        colmin = jnp.full((1, _N), jnp.inf, dtype=jnp.float32)
        d1_pieces = []
        for i in range(_N // _TILE):
            sl = slice(i * _TILE, (i + 1) * _TILE)
            cross = jnp.dot(p[sl, :], gTm2,
                            preferred_element_type=jnp.float32)
            e = cross + np2c[sl, :]
            colmin = jnp.minimum(colmin, jnp.min(e, axis=0, keepdims=True))
            e4 = e + ng2r
            rm = jnp.min(e4, axis=1, keepdims=True)
            d1_pieces.append(jax.lax.transpose(rm, (1, 0)))

        d1sq = jnp.maximum(jnp.concatenate(d1_pieces, axis=1), 1e-12)
        d2sq = jnp.maximum(colmin + ng2r, 1e-12)
        rows.append(jnp.sqrt(d1sq))
        rows.append(jnp.sqrt(d2sq))

    D = jnp.concatenate(rows, axis=0)
    Db = jax.lax.bitcast_convert_type(D, jnp.int32)

    def body(i, m):
        cand = m | jnp.left_shift(jnp.int32(1), jnp.int32(30) - i)
        cnt = jnp.sum(jnp.where(Db >= cand, jnp.int32(1), jnp.int32(0)),
                      axis=1, keepdims=True)
        return jnp.where(cnt >= _K, cand, m)

    m = jax.lax.fori_loop(0, 31, body, jnp.zeros((2 * _B, 1), jnp.int32))
    t = jax.lax.bitcast_convert_type(m, jnp.float32)
    gt_mask = Db > m
    cnt_gt = jnp.sum(jnp.where(gt_mask, jnp.int32(1), jnp.int32(0)),
                     axis=1, keepdims=True)
    sum_gt = jnp.sum(jnp.where(gt_mask, D, jnp.float32(0.0)),
                     axis=1, keepdims=True)
    w = (sum_gt + (jnp.int32(_K) - cnt_gt).astype(jnp.float32) * t) / _K

    s_means = jnp.sum(D, axis=1, keepdims=True) / jnp.float32(_N)
    total = jnp.sum(s_means + _WEIGHT * w)
    out_ref[:, :] = jnp.full((1, 1), 1.0 / _B, jnp.float32) * total


def kernel(pred_pointclouds, gt_pointclouds):
    gtT = jnp.transpose(gt_pointclouds, (0, 2, 1))

    out = pl.pallas_call(
        _chamfer_kernel,
        out_shape=jax.ShapeDtypeStruct((1, 1), jnp.float32),
    )(pred_pointclouds, gtT)
    return out[0, 0]
